# Initial kernel scaffold; baseline (speedup 1.0000x reference)
import jax
import jax.numpy as jnp
from jax.experimental import pallas as pl


def kernel(x, edge_index, Wl0, bl0, Wr0, Wl1, bl1, Wr1, Wl2, bl2, Wr2, Wp1, bp1, Wp2, bp2):
    raise NotImplementedError("write your pallas kernel here")


# speedup vs baseline: 4.6748x; 4.6748x over previous
import functools

import jax
import jax.numpy as jnp
from jax import lax
from jax.experimental import pallas as pl
from jax.experimental.pallas import tpu as pltpu
from jax.experimental.pallas import tpu_sc as plsc

N = 10000
E = 160000
D = 256
H = 128
NS = 16
K = 80
EPT = E // NS
NCH = EPT // K
RPT = 1000
BN = 1000


def _make_agg():
    scratch = [
        pltpu.VMEM((NCH, K), jnp.int32),
        pltpu.VMEM((NCH, K), jnp.int32),
        pltpu.VMEM((K, H), jnp.float32),
        pltpu.VMEM_SHARED((N, H), jnp.float32),
    ]

    def body(hcat, src2, dst2, zrow, agg_o, src_v, dst_v, rows_v, agg_sh):
        c = lax.axis_index("c")
        s = lax.axis_index("s")

        pltpu.sync_copy(src2.at[s], src_v)
        pltpu.sync_copy(dst2.at[s], dst_v)
        @pl.when(s < N // RPT)
        def _():
            pltpu.sync_copy(zrow, agg_sh.at[pl.ds(s * RPT, RPT)])

        delta = c * N

        def adj(j, carry):
            for r in range(K // 16):
                sl = pl.ds(r * 16, 16)
                src_v[j, sl] = src_v[j, sl] + delta
            return carry

        lax.fori_loop(0, NCH, adj, 0)
        plsc.subcore_barrier()

        def step(j, carry):
            pltpu.sync_copy(hcat.at[src_v.at[j]], rows_v)
            pltpu.sync_copy(rows_v, agg_sh.at[dst_v.at[j]], add=True)
            return carry

        lax.fori_loop(0, NCH, step, 0)
        plsc.subcore_barrier()

        @pl.when(s < N // RPT)
        def _():
            off = pl.multiple_of(c * N + s * RPT, RPT)
            pltpu.sync_copy(agg_sh.at[pl.ds(s * RPT, RPT)],
                            agg_o.at[pl.ds(off, RPT)])

    mesh = plsc.VectorSubcoreMesh(core_axis_name="c", subcore_axis_name="s")
    return pl.kernel(body,
                     out_type=jax.ShapeDtypeStruct((2 * N, H), jnp.float32),
                     mesh=mesh, scratch_types=scratch)


def _make_cnt():
    scratch = [
        pltpu.VMEM((NCH, K), jnp.int32),
        pltpu.VMEM((K, H), jnp.float32),
        pltpu.VMEM_SHARED((N, H), jnp.float32),
    ]

    def body(dst2, zrow, ones_h, cnt_o, dst_v, ones_v, cnt_sh):
        c = lax.axis_index("c")
        s = lax.axis_index("s")

        @pl.when(c == 0)
        def _():
            pltpu.sync_copy(dst2.at[s], dst_v)
            pltpu.sync_copy(ones_h, ones_v)
            @pl.when(s < N // RPT)
            def _():
                pltpu.sync_copy(zrow, cnt_sh.at[pl.ds(s * RPT, RPT)])

        plsc.subcore_barrier()

        @pl.when(c == 0)
        def _():
            def step(j, carry):
                pltpu.sync_copy(ones_v, cnt_sh.at[dst_v.at[j]], add=True)
                return carry

            lax.fori_loop(0, NCH, step, 0)

        plsc.subcore_barrier()

        @pl.when((c == 0) & (s < N // RPT))
        def _():
            pltpu.sync_copy(cnt_sh.at[pl.ds(s * RPT, RPT)],
                            cnt_o.at[pl.ds(s * RPT, RPT)])

    mesh = plsc.VectorSubcoreMesh(core_axis_name="c", subcore_axis_name="s")
    return pl.kernel(body,
                     out_type=jax.ShapeDtypeStruct((N, H), jnp.float32),
                     mesh=mesh, scratch_types=scratch)


@functools.cache
def _get_agg():
    return _make_agg()


@functools.cache
def _get_cnt():
    return _make_cnt()


def _mean_halves(aggc_ref, cnt_ref):
    inv = 1.0 / jnp.maximum(cnt_ref[:, 0:1], 1.0)
    return aggc_ref[0] * inv, aggc_ref[1] * inv


def _sage_out(m0, m1, h0, h1, WlT, bl, WrT, lo, hi):
    acc = (jnp.dot(m0, WlT[:H, lo:hi], preferred_element_type=jnp.float32)
           + jnp.dot(m1, WlT[H:, lo:hi], preferred_element_type=jnp.float32)
           + jnp.dot(h0, WrT[:H, lo:hi], preferred_element_type=jnp.float32)
           + jnp.dot(h1, WrT[H:, lo:hi], preferred_element_type=jnp.float32))
    return jnp.maximum(acc + bl[:, lo:hi], 0.0)


def _layer_body(aggc_ref, cnt_ref, hc_ref, WlT_ref, bl_ref, WrT_ref, o_ref):
    m0, m1 = _mean_halves(aggc_ref, cnt_ref)
    h0, h1 = hc_ref[0], hc_ref[1]
    WlT, bl, WrT = WlT_ref[...], bl_ref[...], WrT_ref[...]
    o_ref[0] = _sage_out(m0, m1, h0, h1, WlT, bl, WrT, 0, H)
    o_ref[1] = _sage_out(m0, m1, h0, h1, WlT, bl, WrT, H, D)


def _final_body(aggc_ref, cnt_ref, hc_ref, WlT_ref, bl_ref, WrT_ref,
                Wp1T_ref, bp1_ref, Wp2T_ref, bp2_ref, out_ref):
    m0, m1 = _mean_halves(aggc_ref, cnt_ref)
    h0, h1 = hc_ref[0], hc_ref[1]
    WlT, bl, WrT = WlT_ref[...], bl_ref[...], WrT_ref[...]
    g0 = _sage_out(m0, m1, h0, h1, WlT, bl, WrT, 0, H)
    g1 = _sage_out(m0, m1, h0, h1, WlT, bl, WrT, H, D)
    Wp1T = Wp1T_ref[...]
    p = (jnp.dot(g0, Wp1T[:H, :], preferred_element_type=jnp.float32)
         + jnp.dot(g1, Wp1T[H:, :], preferred_element_type=jnp.float32)
         + bp1_ref[...])
    q = (jnp.dot(p, Wp2T_ref[...], preferred_element_type=jnp.float32)
         + bp2_ref[...])
    m = jnp.max(q, axis=1, keepdims=True)
    lse = m + jnp.log(jnp.sum(jnp.exp(q - m), axis=1, keepdims=True))
    out_ref[...] = q - lse


def _half_spec():
    return pl.BlockSpec((2, BN, H), lambda i: (0, i, 0))


def _node_spec(w):
    return pl.BlockSpec((BN, w), lambda i: (i, 0))


def _full_spec(r, c):
    return pl.BlockSpec((r, c), lambda i: (0, 0))


_layer_tc = pl.pallas_call(
    _layer_body,
    grid=(N // BN,),
    in_specs=[
        _half_spec(), _node_spec(H), _half_spec(),
        _full_spec(D, D), _full_spec(1, D), _full_spec(D, D),
    ],
    out_specs=_half_spec(),
    out_shape=jax.ShapeDtypeStruct((2, N, H), jnp.float32),
)

_final_tc = pl.pallas_call(
    _final_body,
    grid=(N // BN,),
    in_specs=[
        _half_spec(), _node_spec(H), _half_spec(),
        _full_spec(D, D), _full_spec(1, D), _full_spec(D, D),
        _full_spec(D, D), _full_spec(1, D),
        _full_spec(D, H), _full_spec(1, H),
    ],
    out_specs=_node_spec(H),
    out_shape=jax.ShapeDtypeStruct((N, H), jnp.float32),
)


def kernel(x, edge_index, Wl0, bl0, Wr0, Wl1, bl1, Wr1, Wl2, bl2, Wr2,
           Wp1, bp1, Wp2, bp2):
    ei = edge_index.astype(jnp.int32)
    src2 = ei[0].reshape(NS, NCH, K)
    dst2 = ei[1].reshape(NS, NCH, K)
    xcat = jnp.concatenate([x[:, :H], x[:, H:]], axis=0)
    zrow = jnp.zeros((RPT, H), jnp.float32)
    ones_h = jnp.ones((K, H), jnp.float32)

    agg_sc = _get_agg()
    cnt = _get_cnt()(dst2, zrow, ones_h)
    aggc = agg_sc(xcat, src2, dst2, zrow)
    h = _layer_tc(aggc.reshape(2, N, H), cnt, xcat.reshape(2, N, H),
                  Wl0.T, bl0.reshape(1, D), Wr0.T)
    aggc = agg_sc(h.reshape(2 * N, H), src2, dst2, zrow)
    h2 = _layer_tc(aggc.reshape(2, N, H), cnt, h,
                   Wl1.T, bl1.reshape(1, D), Wr1.T)
    aggc = agg_sc(h2.reshape(2 * N, H), src2, dst2, zrow)
    return _final_tc(aggc.reshape(2, N, H), cnt, h2,
                     Wl2.T, bl2.reshape(1, D), Wr2.T,
                     Wp1.T, bp1.reshape(1, D),
                     Wp2.T, bp2.reshape(1, H))
